# Initial kernel scaffold; baseline (speedup 1.0000x reference)
import jax
import jax.numpy as jnp
from jax.experimental import pallas as pl


def kernel(hidden_states, attention_mask, position_ids, ln1_w, q_w, k_w, v_w, o_w, ln2_w, gate_w, w1, w3, w2):
    raise NotImplementedError("write your pallas kernel here")


# speedup vs baseline: 1.5739x; 1.5739x over previous
import functools

import jax
import jax.numpy as jnp
from jax.experimental import pallas as pl
from jax.experimental.pallas import tpu as pltpu

B = 1; S = 2048; D = 1024
NH = 16; NKV = 4; HD = 64
E = 8; TOPK = 2; FF = 2048
EPS = 1e-6; THETA = 1000000.0

NEG = -1e30


_TS1 = 512


def _k1_body(x_ref, ln_ref, qkvw_ref, cos_ref, sin_ref, q_ref, k_ref, v_ref):
    x = x_ref[...]
    var = jnp.mean(jnp.square(x), axis=-1, keepdims=True)
    xn = (x * jax.lax.rsqrt(var + EPS)) * ln_ref[...]
    qkv = jnp.dot(xn, qkvw_ref[...], preferred_element_type=jnp.float32)
    cos = cos_ref[...]
    sin = sin_ref[...]

    def rope(t, nheads):
        t3 = t.reshape(_TS1, nheads, HD)
        rot = jnp.concatenate([-t3[..., HD // 2:], t3[..., :HD // 2]], axis=-1)
        t2 = t3 * cos[:, None, :] + rot * sin[:, None, :]
        return t2.reshape(_TS1, nheads * HD)

    q = qkv[:, :NH * HD]
    k = qkv[:, NH * HD:(NH + NKV) * HD]
    v = qkv[:, (NH + NKV) * HD:]
    q_ref[...] = rope(q, NH)
    k_ref[...] = rope(k, NKV)
    v_ref[...] = v


def _qkv_rope(x, ln1_w, qkv_w, cos, sin):
    grid = (S // _TS1,)
    return pl.pallas_call(
        _k1_body,
        grid=grid,
        in_specs=[
            pl.BlockSpec((_TS1, D), lambda t: (t, 0)),
            pl.BlockSpec((1, D), lambda t: (0, 0)),
            pl.BlockSpec((D, (NH + 2 * NKV) * HD), lambda t: (0, 0)),
            pl.BlockSpec((_TS1, HD), lambda t: (t, 0)),
            pl.BlockSpec((_TS1, HD), lambda t: (t, 0)),
        ],
        out_specs=[
            pl.BlockSpec((_TS1, NH * HD), lambda t: (t, 0)),
            pl.BlockSpec((_TS1, NKV * HD), lambda t: (t, 0)),
            pl.BlockSpec((_TS1, NKV * HD), lambda t: (t, 0)),
        ],
        out_shape=[
            jax.ShapeDtypeStruct((S, NH * HD), jnp.float32),
            jax.ShapeDtypeStruct((S, NKV * HD), jnp.float32),
            jax.ShapeDtypeStruct((S, NKV * HD), jnp.float32),
        ],
    )(x, ln1_w.reshape(1, D), qkv_w, cos, sin)


_QB = 256
_REP = NH // NKV


def _k2_body(q_ref, k_ref, v_ref, o_ref):
    qb = pl.program_id(0)
    row = jax.lax.broadcasted_iota(jnp.int32, (_QB, S), 0) + qb * _QB
    col = jax.lax.broadcasted_iota(jnp.int32, (_QB, S), 1)
    causal = col <= row
    outs = []
    for h in range(NH):
        kv = h // _REP
        qh = q_ref[:, h * HD:(h + 1) * HD]
        kh = k_ref[:, kv * HD:(kv + 1) * HD]
        vh = v_ref[:, kv * HD:(kv + 1) * HD]
        s = jax.lax.dot_general(qh, kh, (((1,), (1,)), ((), ())),
                                preferred_element_type=jnp.float32)
        s = s * (HD ** -0.5)
        s = jnp.where(causal, s, NEG)
        m = jnp.max(s, axis=-1, keepdims=True)
        e = jnp.exp(s - m)
        p = e / jnp.sum(e, axis=-1, keepdims=True)
        outs.append(jnp.dot(p, vh, preferred_element_type=jnp.float32))
    o_ref[...] = jnp.concatenate(outs, axis=1)


def _attention(q, k, v):
    grid = (S // _QB,)
    return pl.pallas_call(
        _k2_body,
        grid=grid,
        in_specs=[
            pl.BlockSpec((_QB, NH * HD), lambda qb: (qb, 0)),
            pl.BlockSpec((S, NKV * HD), lambda qb: (0, 0)),
            pl.BlockSpec((S, NKV * HD), lambda qb: (0, 0)),
        ],
        out_specs=pl.BlockSpec((_QB, NH * HD), lambda qb: (qb, 0)),
        out_shape=jax.ShapeDtypeStruct((S, NH * HD), jnp.float32),
    )(q, k, v)


_TS3 = 512


def _k3_body(a_ref, ow_ref, x_ref, h_ref):
    h_ref[...] = x_ref[...] + jnp.dot(a_ref[...], ow_ref[...],
                                      preferred_element_type=jnp.float32)


def _oproj_residual(attn, o_w, x):
    grid = (S // _TS3,)
    return pl.pallas_call(
        _k3_body,
        grid=grid,
        in_specs=[
            pl.BlockSpec((_TS3, NH * HD), lambda t: (t, 0)),
            pl.BlockSpec((NH * HD, D), lambda t: (0, 0)),
            pl.BlockSpec((_TS3, D), lambda t: (t, 0)),
        ],
        out_specs=pl.BlockSpec((_TS3, D), lambda t: (t, 0)),
        out_shape=jax.ShapeDtypeStruct((S, D), jnp.float32),
    )(attn, o_w, x)


def _k4_body(h_ref, ln_ref, gw_ref, xn_ref, wd_ref):
    h = h_ref[...]
    var = jnp.mean(jnp.square(h), axis=-1, keepdims=True)
    xn = (h * jax.lax.rsqrt(var + EPS)) * ln_ref[...]
    xn_ref[...] = xn
    logits = jax.lax.dot_general(xn, gw_ref[...], (((1,), (0,)), ((), ())),
                                 preferred_element_type=jnp.float32,
                                 precision=jax.lax.Precision.HIGHEST)
    p = jax.nn.softmax(logits, axis=-1)
    idx = jax.lax.broadcasted_iota(jnp.int32, (S, E), 1)
    m0 = jnp.max(p, axis=-1, keepdims=True)
    i0 = jnp.min(jnp.where(p == m0, idx, E), axis=-1, keepdims=True)
    p1 = jnp.where(idx == i0, -1.0, p)
    m1 = jnp.max(p1, axis=-1, keepdims=True)
    i1 = jnp.min(jnp.where(p1 == m1, idx, E), axis=-1, keepdims=True)
    tot = m0 + m1
    wd = jnp.where(idx == i0, m0 / tot, 0.0) + jnp.where(idx == i1, m1 / tot, 0.0)
    wd_ref[...] = wd


def _router(h, ln2_w, gate_w):
    return pl.pallas_call(
        _k4_body,
        grid=(1,),
        in_specs=[
            pl.BlockSpec((S, D), lambda i: (0, 0)),
            pl.BlockSpec((1, D), lambda i: (0, 0)),
            pl.BlockSpec((D, E), lambda i: (0, 0)),
        ],
        out_specs=[
            pl.BlockSpec((S, D), lambda i: (0, 0)),
            pl.BlockSpec((S, E), lambda i: (0, 0)),
        ],
        out_shape=[
            jax.ShapeDtypeStruct((S, D), jnp.float32),
            jax.ShapeDtypeStruct((S, E), jnp.float32),
        ],
    )(h, ln2_w.reshape(1, D), gate_w)


_TS5 = 1024
_FB = 512


def _k5_body(xn_ref, w1_ref, w3_ref, w2_ref, wdT_ref, h_ref, out_ref):
    e = pl.program_id(1)
    f = pl.program_id(2)

    @pl.when((e == 0) & (f == 0))
    def _():
        out_ref[...] = h_ref[...]

    x = xn_ref[...]
    a = jnp.dot(x, w1_ref[0], preferred_element_type=jnp.float32)
    b = jnp.dot(x, w3_ref[0], preferred_element_type=jnp.float32)
    hh = (a * jax.nn.sigmoid(a)) * b
    y = jnp.dot(hh, w2_ref[0], preferred_element_type=jnp.float32)
    out_ref[...] += y * wdT_ref[0, 0][:, None]


def _moe_dense(xn, w1, w3, w2, wdense, h):
    wdT = wdense.T.reshape(E, 1, S)
    grid = (S // _TS5, E, FF // _FB)
    return pl.pallas_call(
        _k5_body,
        grid=grid,
        in_specs=[
            pl.BlockSpec((_TS5, D), lambda t, e, f: (t, 0)),
            pl.BlockSpec((1, D, _FB), lambda t, e, f: (e, 0, f)),
            pl.BlockSpec((1, D, _FB), lambda t, e, f: (e, 0, f)),
            pl.BlockSpec((1, _FB, D), lambda t, e, f: (e, f, 0)),
            pl.BlockSpec((1, 1, _TS5), lambda t, e, f: (e, 0, t)),
            pl.BlockSpec((_TS5, D), lambda t, e, f: (t, 0)),
        ],
        out_specs=pl.BlockSpec((_TS5, D), lambda t, e, f: (t, 0)),
        out_shape=jax.ShapeDtypeStruct((S, D), jnp.float32),
    )(xn, w1, w3, w2, wdT, h)


def kernel(hidden_states, attention_mask, position_ids, ln1_w, q_w, k_w, v_w,
           o_w, ln2_w, gate_w, w1, w3, w2):
    del attention_mask
    x = hidden_states.reshape(S, D)
    pos = position_ids.reshape(S).astype(jnp.float32)

    inv = 1.0 / (THETA ** (jnp.arange(0, HD, 2, dtype=jnp.float32) / HD))
    ang = pos[:, None] * inv[None, :]
    cos = jnp.concatenate([jnp.cos(ang), jnp.cos(ang)], axis=-1)
    sin = jnp.concatenate([jnp.sin(ang), jnp.sin(ang)], axis=-1)

    qkv_w = jnp.concatenate([q_w, k_w, v_w], axis=1)
    q, k, v = _qkv_rope(x, ln1_w, qkv_w, cos, sin)
    attn = _attention(q, k, v)
    h = _oproj_residual(attn, o_w, x)
    xn2, wdense = _router(h, ln2_w, gate_w)
    out = _moe_dense(xn2, w1, w3, w2, wdense, h)
    return out.reshape(B, S, D)
